# Initial kernel scaffold; baseline (speedup 1.0000x reference)
import jax
import jax.numpy as jnp
from jax.experimental import pallas as pl


def kernel(x0, x1, x2, up_index_0, up_index_1, boundary_src_1, boundary_dst_1, boundary_src_2, boundary_dst_2, mask, init_W, init_b, lin1_W, lin1_b, l0_up1_W, l0_up1_b, l0_up2_W, l0_up2_b, l0_bd1_W, l0_bd1_b, l0_bd2_W, l0_bd2_b, l0_comb_W, l0_comb_b, l1_up1_W, l1_up1_b, l1_up2_W, l1_up2_b, l1_bd1_W, l1_bd1_b, l1_bd2_W, l1_bd2_b, l1_comb_W, l1_comb_b):
    raise NotImplementedError("write your pallas kernel here")


# speedup vs baseline: 5.1187x; 5.1187x over previous
import functools

import jax
import jax.numpy as jnp
from jax import lax
from jax.experimental import pallas as pl
from jax.experimental.pallas import tpu as pltpu
from jax.experimental.pallas import tpu_sc as plsc

N0 = 10000
E0 = 320000
HID = 64

_NC, _NS = 2, 16
_NW = _NC * _NS
_K = 128
_CHUNKS_PER_W = -(-E0 // (_K * _NW))
_EPAD = _CHUNKS_PER_W * _K * _NW
_EV_PER_W = _CHUNKS_PER_W * _K
_NTRASH = 16
_NACC = 10112
_ROWS_PER_TILE = _NACC // _NS


@functools.cache
def _make_segsum(d):
    mesh = plsc.VectorSubcoreMesh(core_axis_name="c", subcore_axis_name="s")

    @functools.partial(
        pl.kernel,
        mesh=mesh,
        out_type=jax.ShapeDtypeStruct((_NC, _NACC, d), jnp.float32),
        scratch_types=[
            pltpu.VMEM_SHARED((_NACC, d), jnp.float32),
            pltpu.VMEM((_K,), jnp.int32),
            pltpu.VMEM((_K,), jnp.int32),
            pltpu.VMEM((_K, d), jnp.float32),
            pltpu.SemaphoreType.DMA,
        ],
    )
    def seg(table_hbm, src_hbm, dst_hbm, zeros_hbm, out_hbm,
            acc_s, idx_v, dst_v, rows_v, sem):
        c = lax.axis_index("c")
        s = lax.axis_index("s")
        w = c * _NS + s

        pltpu.sync_copy(zeros_hbm, acc_s.at[pl.ds(s * _ROWS_PER_TILE,
                                                  _ROWS_PER_TILE)])
        plsc.subcore_barrier()

        def body(i, carry):
            e0 = w * _EV_PER_W + i * _K
            pltpu.sync_copy(src_hbm.at[pl.ds(e0, _K)], idx_v)
            pltpu.sync_copy(dst_hbm.at[pl.ds(e0, _K)], dst_v)
            pltpu.async_copy(table_hbm.at[idx_v], rows_v, sem).wait()
            pltpu.sync_copy(rows_v, acc_s.at[dst_v], add=True)
            return carry

        lax.fori_loop(0, _CHUNKS_PER_W, body, 0)
        plsc.subcore_barrier()

        r0 = s * _ROWS_PER_TILE
        pltpu.sync_copy(acc_s.at[pl.ds(r0, _ROWS_PER_TILE)],
                        out_hbm.at[c, pl.ds(r0, _ROWS_PER_TILE)])

    return seg


def _relu(x):
    return jnp.maximum(x, 0.0)


def _dot(a, b):
    return jnp.dot(a, b, preferred_element_type=jnp.float32)


_RB = 1000
_GRID = N0 // _RB


def _full(shape):
    return pl.BlockSpec(shape, lambda i: tuple(0 for _ in shape))


def _rows(d):
    return pl.BlockSpec((_RB, d), lambda i: (i, 0))


def _init_body(x_ref, w_ref, b_ref, o_ref):
    o_ref[...] = _dot(x_ref[...], w_ref[...]) + b_ref[...]


def _tc_init(x0, w, b):
    return pl.pallas_call(
        _init_body,
        grid=(_GRID,),
        in_specs=[_rows(128), _full((128, 128)), _full((1, 128))],
        out_specs=_rows(128),
        out_shape=jax.ShapeDtypeStruct((N0, 128), jnp.float32),
    )(x0, w, b.reshape(1, 128))


def _layer_body(fi, wout, p0, p1, base, w1u, b1u, w2u, b2u, w1b, b1b,
                w2b, b2b, cwu, cwb, cb, o_ref):
    b = base[...][:, :fi]
    a = b + p0[...][:, :fi] + p1[...][:, :fi]
    hu = _relu(_dot(a, w1u[...]) + b1u[...])
    hu = _relu(_dot(hu, w2u[...]) + b2u[...])
    hb = _relu(_dot(b, w1b[...]) + b1b[...])
    hb = _relu(_dot(hb, w2b[...]) + b2b[...])
    o = _relu(_dot(hu, cwu[...]) + _dot(hb, cwb[...]) + cb[...])
    if wout > HID:
        o = jnp.concatenate([o, jnp.zeros((o.shape[0], wout - HID),
                                          jnp.float32)], axis=1)
    o_ref[...] = o


def _tc_layer(p0, p1, base, fi, win, wout, w1u, b1u, w2u, b2u, w1b, b1b,
              w2b, b2b, cw, cb):
    args = (p0, p1, base,
            w1u, b1u.reshape(1, HID), w2u, b2u.reshape(1, HID),
            w1b, b1b.reshape(1, HID), w2b, b2b.reshape(1, HID),
            cw[:HID], cw[HID:], cb.reshape(1, HID))
    return pl.pallas_call(
        functools.partial(_layer_body, fi, wout),
        grid=(_GRID,),
        in_specs=[_rows(win), _rows(win), _rows(win),
                  _full((fi, HID)), _full((1, HID)),
                  _full((HID, HID)), _full((1, HID)),
                  _full((fi, HID)), _full((1, HID)),
                  _full((HID, HID)), _full((1, HID)),
                  _full((HID, HID)), _full((HID, HID)), _full((1, HID))],
        out_specs=_rows(wout),
        out_shape=jax.ShapeDtypeStruct((N0, wout), jnp.float32),
    )(*args)


def _readout_body(x_ref, m_ref, w_ref, b_ref, o_ref):
    o_ref[...] = _dot(x_ref[...] * m_ref[...], w_ref[...]) + b_ref[...]


def _tc_readout(x, maskf, w, b, ncls):
    wp = jnp.zeros((HID, 128), jnp.float32).at[:, :ncls].set(w)
    bp = jnp.zeros((1, 128), jnp.float32).at[0, :ncls].set(b)
    out = pl.pallas_call(
        _readout_body,
        grid=(_GRID,),
        in_specs=[_rows(HID), _rows(1), _full((HID, 128)), _full((1, 128))],
        out_specs=_rows(128),
        out_shape=jax.ShapeDtypeStruct((N0, 128), jnp.float32),
    )(x, maskf, wp, bp)
    return out[:, :ncls]


def kernel(x0, x1, x2, up_index_0, up_index_1, boundary_src_1,
           boundary_dst_1, boundary_src_2, boundary_dst_2, mask,
           init_W, init_b, lin1_W, lin1_b,
           l0_up1_W, l0_up1_b, l0_up2_W, l0_up2_b,
           l0_bd1_W, l0_bd1_b, l0_bd2_W, l0_bd2_b,
           l0_comb_W, l0_comb_b,
           l1_up1_W, l1_up1_b, l1_up2_W, l1_up2_b,
           l1_bd1_W, l1_bd1_b, l1_bd2_W, l1_bd2_b,
           l1_comb_W, l1_comb_b):
    src = up_index_0[0].astype(jnp.int32)
    dst = up_index_0[1].astype(jnp.int32)
    npad = _EPAD - E0
    pad_src = (jnp.arange(npad, dtype=jnp.int32) * 37) % N0
    pad_dst = N0 + (jnp.arange(npad, dtype=jnp.int32) % _NTRASH)
    src_p = jnp.concatenate([src, pad_src])
    dst_p = jnp.concatenate([dst, pad_dst])
    z128 = jnp.zeros((_ROWS_PER_TILE, 128), jnp.float32)

    x = _tc_init(x0, init_W, init_b)

    p = _make_segsum(128)(x, src_p, dst_p, z128)
    x = _tc_layer(p[0, :N0], p[1, :N0], x, 128, 128, 128,
                  l0_up1_W[0], l0_up1_b[0], l0_up2_W[0], l0_up2_b[0],
                  l0_bd1_W[0], l0_bd1_b[0], l0_bd2_W[0], l0_bd2_b[0],
                  l0_comb_W[0], l0_comb_b[0])

    p = _make_segsum(128)(x, src_p, dst_p, z128)
    x = _tc_layer(p[0, :N0], p[1, :N0], x, HID, 128, HID,
                  l1_up1_W[0], l1_up1_b[0], l1_up2_W[0], l1_up2_b[0],
                  l1_bd1_W[0], l1_bd1_b[0], l1_bd2_W[0], l1_bd2_b[0],
                  l1_comb_W[0], l1_comb_b[0])

    maskf = mask.astype(jnp.float32).reshape(N0, 1)
    return _tc_readout(x, maskf, lin1_W, lin1_b, lin1_W.shape[1])


# speedup vs baseline: 9.9908x; 1.9518x over previous
import functools

import jax
import jax.numpy as jnp
from jax import lax
from jax.experimental import pallas as pl
from jax.experimental.pallas import tpu as pltpu
from jax.experimental.pallas import tpu_sc as plsc

N0 = 10000
E0 = 320000
HID = 64

_NC, _NS = 2, 16
_NW = _NC * _NS
_K = 128
_CHUNKS_PER_W = 80
_EPAD = _CHUNKS_PER_W * _K * _NW
_EV_PER_W = _CHUNKS_PER_W * _K
_NTRASH = 16
_NACC = 10112
_ROWS_PER_TILE = _NACC // _NS


@functools.cache
def _make_segsum(d):
    mesh = plsc.VectorSubcoreMesh(core_axis_name="c", subcore_axis_name="s")

    @functools.partial(
        pl.kernel,
        mesh=mesh,
        out_type=jax.ShapeDtypeStruct((_NC, N0, d), jnp.float32),
        scratch_types=[
            pltpu.VMEM_SHARED((_NACC, d), jnp.float32),
            pltpu.VMEM((_CHUNKS_PER_W // 2, _K), jnp.int32),
            pltpu.VMEM((_CHUNKS_PER_W // 2, _K), jnp.int32),
            pltpu.VMEM((_K, d), jnp.float32),
            pltpu.VMEM((_K, d), jnp.float32),
            pltpu.SemaphoreType.DMA,
            pltpu.SemaphoreType.DMA,
        ],
    )
    def seg(table_hbm, src_hbm, dst_hbm, zeros_hbm, out_hbm,
            acc_s, src_v, dst_v, rows_a, rows_b, sem_a, sem_b):
        c = lax.axis_index("c")
        s = lax.axis_index("s")
        w = c * _NS + s
        half = _CHUNKS_PER_W // 2

        pltpu.sync_copy(zeros_hbm, acc_s.at[pl.ds(s * _ROWS_PER_TILE,
                                                  _ROWS_PER_TILE)])
        plsc.subcore_barrier()

        def gather(i, buf, sem):
            return pltpu.make_async_copy(table_hbm.at[src_v.at[i]], buf, sem)

        for h in range(2):
            row0 = w * _CHUNKS_PER_W + h * half
            pltpu.sync_copy(src_hbm.at[pl.ds(row0, half)], src_v)
            pltpu.sync_copy(dst_hbm.at[pl.ds(row0, half)], dst_v)
            gather(0, rows_a, sem_a).start()

            def body(j, carry):
                i0 = j * 2
                gather(i0 + 1, rows_b, sem_b).start()
                gather(i0, rows_a, sem_a).wait()
                pltpu.sync_copy(rows_a, acc_s.at[dst_v.at[i0]], add=True)

                @pl.when(i0 + 2 < half)
                def _():
                    gather(i0 + 2, rows_a, sem_a).start()

                gather(i0 + 1, rows_b, sem_b).wait()
                pltpu.sync_copy(rows_b, acc_s.at[dst_v.at[i0 + 1]], add=True)
                return carry

            lax.fori_loop(0, half // 2, body, 0)
        plsc.subcore_barrier()

        r0 = s * _ROWS_PER_TILE
        last_rows = N0 - (_NS - 1) * _ROWS_PER_TILE

        @pl.when(s < _NS - 1)
        def _():
            pltpu.sync_copy(acc_s.at[pl.ds(r0, _ROWS_PER_TILE)],
                            out_hbm.at[c, pl.ds(r0, _ROWS_PER_TILE)])

        @pl.when(s == _NS - 1)
        def _():
            pltpu.sync_copy(acc_s.at[pl.ds(r0, last_rows)],
                            out_hbm.at[c, pl.ds(r0, last_rows)])

    return seg


def _relu(x):
    return jnp.maximum(x, 0.0)


def _dot(a, b):
    return jnp.dot(a, b, preferred_element_type=jnp.float32)


_RB = 1000
_GRID = N0 // _RB


def _full(shape):
    return pl.BlockSpec(shape, lambda i: tuple(0 for _ in shape))


def _rows(d):
    return pl.BlockSpec((_RB, d), lambda i: (i, 0))


def _init_body(x_ref, w_ref, b_ref, o_ref):
    o_ref[...] = _dot(x_ref[...], w_ref[...]) + b_ref[...]


def _tc_init(x0, w, b):
    return pl.pallas_call(
        _init_body,
        grid=(_GRID,),
        in_specs=[_rows(128), _full((128, 128)), _full((1, 128))],
        out_specs=_rows(128),
        out_shape=jax.ShapeDtypeStruct((N0, 128), jnp.float32),
    )(x0, w, b.reshape(1, 128))


def _prow(win):
    def im0(i):
        return (0, i, 0)

    def im1(i):
        return (1, i, 0)

    return (pl.BlockSpec((1, _RB, win), im0),
            pl.BlockSpec((1, _RB, win), im1))


def _layer_body(fi, wout, readout, p0, p1, base, w1u, b1u, w2u, b2u,
                w1b, b1b, w2b, b2b, cwu, cwb, cb, *rest):
    o_ref = rest[-1]
    b = base[...][:, :fi]
    a = b + p0[0][:, :fi] + p1[0][:, :fi]
    hu = _relu(_dot(a, w1u[...]) + b1u[...])
    hu = _relu(_dot(hu, w2u[...]) + b2u[...])
    hb = _relu(_dot(b, w1b[...]) + b1b[...])
    hb = _relu(_dot(hb, w2b[...]) + b2b[...])
    o = _relu(_dot(hu, cwu[...]) + _dot(hb, cwb[...]) + cb[...])
    if readout:
        m_ref, wp_ref, bp_ref = rest[:3]
        o = _dot(o * m_ref[...], wp_ref[...]) + bp_ref[...]
    elif wout > HID:
        o = jnp.concatenate([o, jnp.zeros((o.shape[0], wout - HID),
                                          jnp.float32)], axis=1)
    o_ref[...] = o


def _tc_layer(p, base, fi, win, wout, w1u, b1u, w2u, b2u, w1b, b1b,
              w2b, b2b, cw, cb, readout=None):
    spec0, spec1 = _prow(win)
    args = [p, p, base,
            w1u, b1u.reshape(1, HID), w2u, b2u.reshape(1, HID),
            w1b, b1b.reshape(1, HID), w2b, b2b.reshape(1, HID),
            cw[:HID], cw[HID:], cb.reshape(1, HID)]
    specs = [spec0, spec1, _rows(win),
             _full((fi, HID)), _full((1, HID)),
             _full((HID, HID)), _full((1, HID)),
             _full((fi, HID)), _full((1, HID)),
             _full((HID, HID)), _full((1, HID)),
             _full((HID, HID)), _full((HID, HID)), _full((1, HID))]
    if readout is not None:
        maskf, w, b, ncls = readout
        wp = jnp.zeros((HID, 128), jnp.float32).at[:, :ncls].set(w)
        bp = jnp.zeros((1, 128), jnp.float32).at[0, :ncls].set(b)
        args += [maskf, wp, bp]
        specs += [_rows(1), _full((HID, 128)), _full((1, 128))]
        wout = 128
    return pl.pallas_call(
        functools.partial(_layer_body, fi, wout, readout is not None),
        grid=(_GRID,),
        in_specs=specs,
        out_specs=_rows(wout),
        out_shape=jax.ShapeDtypeStruct((N0, wout), jnp.float32),
    )(*args)


def kernel(x0, x1, x2, up_index_0, up_index_1, boundary_src_1,
           boundary_dst_1, boundary_src_2, boundary_dst_2, mask,
           init_W, init_b, lin1_W, lin1_b,
           l0_up1_W, l0_up1_b, l0_up2_W, l0_up2_b,
           l0_bd1_W, l0_bd1_b, l0_bd2_W, l0_bd2_b,
           l0_comb_W, l0_comb_b,
           l1_up1_W, l1_up1_b, l1_up2_W, l1_up2_b,
           l1_bd1_W, l1_bd1_b, l1_bd2_W, l1_bd2_b,
           l1_comb_W, l1_comb_b):
    src = up_index_0[0].astype(jnp.int32)
    dst = up_index_0[1].astype(jnp.int32)
    npad = _EPAD - E0
    pad_src = (jnp.arange(npad, dtype=jnp.int32) * 37) % N0
    pad_dst = N0 + (jnp.arange(npad, dtype=jnp.int32) % _NTRASH)
    src_p = jnp.concatenate([src, pad_src]).reshape(_EPAD // _K, _K)
    dst_p = jnp.concatenate([dst, pad_dst]).reshape(_EPAD // _K, _K)
    z128 = jnp.zeros((_ROWS_PER_TILE, 128), jnp.float32)

    x = _tc_init(x0, init_W, init_b)

    p = _make_segsum(128)(x, src_p, dst_p, z128)
    x = _tc_layer(p, x, 128, 128, 128,
                  l0_up1_W[0], l0_up1_b[0], l0_up2_W[0], l0_up2_b[0],
                  l0_bd1_W[0], l0_bd1_b[0], l0_bd2_W[0], l0_bd2_b[0],
                  l0_comb_W[0], l0_comb_b[0])

    p = _make_segsum(128)(x, src_p, dst_p, z128)
    maskf = mask.astype(jnp.float32).reshape(N0, 1)
    out = _tc_layer(p, x, HID, 128, HID,
                    l1_up1_W[0], l1_up1_b[0], l1_up2_W[0], l1_up2_b[0],
                    l1_bd1_W[0], l1_bd1_b[0], l1_bd2_W[0], l1_bd2_b[0],
                    l1_comb_W[0], l1_comb_b[0],
                    readout=(maskf, lin1_W, lin1_b, lin1_W.shape[1]))
    return out[:, :lin1_W.shape[1]]
